# Initial kernel scaffold; baseline (speedup 1.0000x reference)
import jax
import jax.numpy as jnp
from jax.experimental import pallas as pl


def kernel(text_embeds, video_embeds):
    raise NotImplementedError("write your pallas kernel here")


# speedup vs baseline: 26.2520x; 26.2520x over previous
import functools

import jax
import jax.numpy as jnp
from jax.experimental import pallas as pl
from jax.experimental.pallas import tpu as pltpu

K = 4
VB = 8


def _body(text_ref, vid_ref, pooled_ref, idx_ref):
    text = text_ref[...]
    vid = vid_ref[...]
    vb, F, D = vid.shape
    T = text.shape[0]

    sims = jax.lax.dot_general(
        vid.reshape(vb * F, D), text,
        (((1,), (1,)), ((), ())),
        preferred_element_type=jnp.float32,
    ).reshape(vb, F, T)

    f_iota = jax.lax.broadcasted_iota(jnp.int32, (vb, F, T), 1)
    cur = sims
    idxs = []
    for j in range(K):
        m = jnp.max(cur, axis=1, keepdims=True)
        idx_j = jnp.min(jnp.where(cur == m, f_iota, F), axis=1)
        idxs.append(idx_j)
        cur = jnp.where(f_iota == idx_j[:, None, :], -jnp.inf, cur)

    idx = jnp.stack(idxs, axis=1)
    idx_ref[...] = jnp.broadcast_to(idx[:, :, None, :], (vb, K, D, T))

    t_f_iota = jax.lax.broadcasted_iota(jnp.int32, (vb, T, F), 2)
    oh = jnp.zeros((vb, T, F), jnp.float32)
    for j in range(K):
        oh = oh + (t_f_iota == idxs[j][:, :, None]).astype(jnp.float32)
    for v in range(vb):
        pooled_ref[v] = jax.lax.dot_general(
            oh[v], vid[v], (((1,), (0,)), ((), ())),
            preferred_element_type=jnp.float32,
        )


@jax.jit
def kernel(text_embeds, video_embeds):
    T, D = text_embeds.shape
    V, F, _ = video_embeds.shape
    grid = (V // VB,)
    pooled, idx_exp = pl.pallas_call(
        _body,
        grid=grid,
        in_specs=[
            pl.BlockSpec((T, D), lambda i: (0, 0)),
            pl.BlockSpec((VB, F, D), lambda i: (i, 0, 0)),
        ],
        out_specs=[
            pl.BlockSpec((VB, T, D), lambda i: (i, 0, 0)),
            pl.BlockSpec((VB, K, D, T), lambda i: (i, 0, 0, 0)),
        ],
        out_shape=[
            jax.ShapeDtypeStruct((V, T, D), jnp.float32),
            jax.ShapeDtypeStruct((V, K, D, T), jnp.int32),
        ],
    )(text_embeds, video_embeds)
    return pooled, idx_exp
